# Initial kernel scaffold; baseline (speedup 1.0000x reference)
import jax
import jax.numpy as jnp
from jax.experimental import pallas as pl


def kernel(x, edge_index, filter_coeff, W, b):
    raise NotImplementedError("write your pallas kernel here")


# speedup vs baseline: 9.9078x; 9.9078x over previous
import functools

import jax
import jax.numpy as jnp
from jax import lax
from jax.experimental import pallas as pl
from jax.experimental.pallas import tpu as pltpu
from jax.experimental.pallas import tpu_sc as plsc

N = 10000
D = 128
K = 3
E = 320000

NC = 2
NS = 16
NW = NC * NS
NP = 10240
RPT = NP // NW
EPW = E // NW
C = 80
CH = EPW // C
DEGW = 16

_mesh = plsc.VectorSubcoreMesh(
    core_axis_name="c", subcore_axis_name="s", num_cores=NC, num_subcores=NS
)


@functools.partial(
    pl.kernel,
    out_type=jax.ShapeDtypeStruct((NC, NP, DEGW), jnp.float32),
    mesh=_mesh,
    scratch_types=[
        pltpu.VMEM((C,), jnp.int32),
        pltpu.VMEM((C, DEGW), jnp.float32),
        pltpu.VMEM((RPT, DEGW), jnp.float32),
        pltpu.VMEM_SHARED((NP, DEGW), jnp.float32),
    ],
)
def _deg_kernel(dst_hbm, out_hbm, dstv, onesv, zb, acc):
    c = lax.axis_index("c")
    s = lax.axis_index("s")
    wid = c * NS + s

    def fill(i, _):
        zb[i, pl.ds(0, 16)] = jnp.zeros((16,), jnp.float32)
        return 0

    lax.fori_loop(0, RPT, fill, 0)

    def fill1(i, _):
        onesv[i, pl.ds(0, 16)] = jnp.ones((16,), jnp.float32)
        return 0

    lax.fori_loop(0, C, fill1, 0)

    pltpu.sync_copy(zb, acc.at[pl.ds(s * RPT, RPT), :])
    plsc.subcore_barrier()

    ebase = wid * EPW

    def body(j, _):
        base = pl.multiple_of(ebase + j * C, C)
        pltpu.sync_copy(dst_hbm.at[pl.ds(base, C)], dstv)
        pltpu.sync_copy(onesv, acc.at[dstv], add=True)
        return 0

    lax.fori_loop(0, CH, body, 0)
    plsc.subcore_barrier()
    pltpu.sync_copy(
        acc.at[pl.ds(s * RPT, RPT), :], out_hbm.at[c, pl.ds(s * RPT, RPT), :]
    )


@functools.partial(
    pl.kernel,
    out_type=jax.ShapeDtypeStruct((NC, NP, D), jnp.float32),
    mesh=_mesh,
    scratch_types=[
        pltpu.VMEM((C,), jnp.int32),
        pltpu.VMEM((C,), jnp.int32),
        pltpu.VMEM((C, D), jnp.float32),
        pltpu.VMEM((128, D), jnp.float32),
        pltpu.VMEM_SHARED((NP, D), jnp.float32),
        pltpu.SemaphoreType.DMA,
    ],
)
def _spmv_kernel(src_hbm, dst_hbm, tab_hbm, out_hbm, srcv, dstv, rows, zb, acc, sem):
    c = lax.axis_index("c")
    s = lax.axis_index("s")
    wid = c * NS + s

    def fill(i, _):
        for k in range(D // 16):
            zb[i, pl.ds(k * 16, 16)] = jnp.zeros((16,), jnp.float32)
        return 0

    lax.fori_loop(0, 128, fill, 0)

    for t in range(RPT // 128):
        pltpu.sync_copy(zb, acc.at[pl.ds(s * RPT + t * 128, 128), :])
    plsc.subcore_barrier()

    ebase = wid * EPW

    def body(j, _):
        base = pl.multiple_of(ebase + j * C, C)
        pltpu.sync_copy(src_hbm.at[pl.ds(base, C)], srcv)
        pltpu.sync_copy(dst_hbm.at[pl.ds(base, C)], dstv)
        pltpu.async_copy(tab_hbm.at[srcv], rows, sem).wait()
        pltpu.sync_copy(rows, acc.at[dstv], add=True)
        return 0

    lax.fori_loop(0, CH, body, 0)
    plsc.subcore_barrier()
    for t in range(RPT // 128):
        pltpu.sync_copy(
            acc.at[pl.ds(s * RPT + t * 128, 128), :],
            out_hbm.at[c, pl.ds(s * RPT + t * 128, 128), :],
        )


BR = 1024


def _scale_body(degp_ref, x_ref, dinv_ref, h0_ref):
    deg = degp_ref[0, :, 0:1] + degp_ref[1, :, 0:1]
    dinv = lax.rsqrt(jnp.maximum(deg, 1.0))
    dinv_ref[...] = dinv
    h0_ref[...] = x_ref[...] * dinv


def _combine_body(q_ref, dinv_ref, tx1_ref, h1_ref):
    dinv = dinv_ref[...]
    tx1 = -dinv * (q_ref[0] + q_ref[1])
    tx1_ref[...] = tx1
    h1_ref[...] = dinv * tx1


def _final_body(x_ref, tx1_ref, r_ref, dinv_ref, fc_ref, w_ref, b_ref, out_ref):
    tx2 = -2.0 * dinv_ref[...] * (r_ref[0] + r_ref[1]) - x_ref[...]
    acc = jnp.dot(fc_ref[:, 0:1] * x_ref[...], w_ref[0],
                  preferred_element_type=jnp.float32)
    acc = acc + jnp.dot(fc_ref[:, 1:2] * tx1_ref[...], w_ref[1],
                        preferred_element_type=jnp.float32)
    acc = acc + jnp.dot(fc_ref[:, 2:3] * tx2, w_ref[2],
                        preferred_element_type=jnp.float32)
    out_ref[...] = acc + b_ref[...]


def _row_spec(width):
    return pl.BlockSpec((BR, width), lambda i: (i, 0))


def _part_spec(width):
    return pl.BlockSpec((NC, BR, width), lambda i: (0, i, 0))


_scale_call = pl.pallas_call(
    _scale_body,
    grid=(NP // BR,),
    in_specs=[_part_spec(DEGW), _row_spec(D)],
    out_specs=[_row_spec(1), _row_spec(D)],
    out_shape=[
        jax.ShapeDtypeStruct((NP, 1), jnp.float32),
        jax.ShapeDtypeStruct((NP, D), jnp.float32),
    ],
)

_combine_call = pl.pallas_call(
    _combine_body,
    grid=(NP // BR,),
    in_specs=[_part_spec(D), _row_spec(1)],
    out_specs=[_row_spec(D), _row_spec(D)],
    out_shape=[
        jax.ShapeDtypeStruct((NP, D), jnp.float32),
        jax.ShapeDtypeStruct((NP, D), jnp.float32),
    ],
)

_final_call = pl.pallas_call(
    _final_body,
    grid=(NP // BR,),
    in_specs=[
        _row_spec(D),
        _row_spec(D),
        _part_spec(D),
        _row_spec(1),
        _row_spec(K),
        pl.BlockSpec((K, D, D), lambda i: (0, 0, 0)),
        pl.BlockSpec((1, D), lambda i: (0, 0)),
    ],
    out_specs=_row_spec(D),
    out_shape=jax.ShapeDtypeStruct((NP, D), jnp.float32),
)


def kernel(x, edge_index, filter_coeff, W, b):
    src = edge_index[0].astype(jnp.int32)
    dst = edge_index[1].astype(jnp.int32)
    x_p = jnp.pad(x, ((0, NP - N), (0, 0)))
    fc_p = jnp.pad(filter_coeff.T, ((0, NP - N), (0, 0)))

    degp = _deg_kernel(dst)
    dinv, h0 = _scale_call(degp, x_p)
    q = _spmv_kernel(src, dst, h0)
    tx1, h1 = _combine_call(q, dinv)
    r = _spmv_kernel(src, dst, h1)
    out_p = _final_call(x_p, tx1, r, dinv, fc_p, W, b[None, :])
    return out_p[:N]


# speedup vs baseline: 15.9404x; 1.6089x over previous
import functools

import jax
import jax.numpy as jnp
from jax import lax
from jax.experimental import pallas as pl
from jax.experimental.pallas import tpu as pltpu
from jax.experimental.pallas import tpu_sc as plsc

N = 10000
D = 128
K = 3
E = 320000

NC = 2
NS = 16
NW = NC * NS
NP = 10240
RPT = NP // NW
EPW = E // NW
C = 80
CH = EPW // C
DEGW = 16

_mesh = plsc.VectorSubcoreMesh(
    core_axis_name="c", subcore_axis_name="s", num_cores=NC, num_subcores=NS
)


@functools.partial(
    pl.kernel,
    out_type=jax.ShapeDtypeStruct((NC, NP, DEGW), jnp.float32),
    mesh=_mesh,
    scratch_types=[
        pltpu.VMEM((C,), jnp.int32),
        pltpu.VMEM((C,), jnp.int32),
        pltpu.VMEM((C, DEGW), jnp.float32),
        pltpu.VMEM((RPT, DEGW), jnp.float32),
        pltpu.VMEM_SHARED((NP, DEGW), jnp.float32),
        pltpu.SemaphoreType.DMA,
        pltpu.SemaphoreType.DMA,
    ],
)
def _deg_kernel(dst_hbm, out_hbm, dstv0, dstv1, onesv, zb, acc, sem0, sem1):
    c = lax.axis_index("c")
    s = lax.axis_index("s")
    wid = c * NS + s

    def fill(i, _):
        zb[i, pl.ds(0, 16)] = jnp.zeros((16,), jnp.float32)
        return 0

    lax.fori_loop(0, RPT, fill, 0)

    def fill1(i, _):
        onesv[i, pl.ds(0, 16)] = jnp.ones((16,), jnp.float32)
        return 0

    lax.fori_loop(0, C, fill1, 0)

    pltpu.sync_copy(zb, acc.at[pl.ds(s * RPT, RPT), :])
    plsc.subcore_barrier()

    ebase = wid * EPW
    sems = (sem0, sem1)
    dstv = (dstv0, dstv1)

    def start(j, b):
        base = pl.multiple_of(ebase + j * C, C)
        pltpu.async_copy(dst_hbm.at[pl.ds(base, C)], dstv[b], sems[b])

    def finish(j, b):
        pltpu.make_async_copy(
            dst_hbm.at[pl.ds(0, C)], dstv[b], sems[b]
        ).wait()
        pltpu.sync_copy(onesv, acc.at[dstv[b]], add=True)

    start(0, 0)

    def body(jj, _):
        j0 = 2 * jj
        start(j0 + 1, 1)
        finish(j0, 0)
        start(j0 + 2, 0)
        finish(j0 + 1, 1)
        return 0

    lax.fori_loop(0, (CH - 3) // 2, body, 0)
    j0 = CH - 3
    start(j0 + 1, 1)
    finish(j0, 0)
    start(j0 + 2, 0)
    finish(j0 + 1, 1)
    finish(j0 + 2, 0)

    plsc.subcore_barrier()
    pltpu.sync_copy(
        acc.at[pl.ds(s * RPT, RPT), :], out_hbm.at[c, pl.ds(s * RPT, RPT), :]
    )


@functools.partial(
    pl.kernel,
    out_type=jax.ShapeDtypeStruct((NC, NP, D), jnp.float32),
    mesh=_mesh,
    scratch_types=[
        pltpu.VMEM((C,), jnp.int32),
        pltpu.VMEM((C,), jnp.int32),
        pltpu.VMEM((C,), jnp.int32),
        pltpu.VMEM((C,), jnp.int32),
        pltpu.VMEM((C, D), jnp.float32),
        pltpu.VMEM((C, D), jnp.float32),
        pltpu.VMEM((128, D), jnp.float32),
        pltpu.VMEM_SHARED((NP, D), jnp.float32),
        pltpu.SemaphoreType.DMA,
        pltpu.SemaphoreType.DMA,
    ],
)
def _spmv_kernel(src_hbm, dst_hbm, tab_hbm, out_hbm, srcv0, srcv1, dstv0, dstv1,
                 rows0, rows1, zb, acc, sem0, sem1):
    c = lax.axis_index("c")
    s = lax.axis_index("s")
    wid = c * NS + s

    def fill(i, _):
        for k in range(D // 16):
            zb[i, pl.ds(k * 16, 16)] = jnp.zeros((16,), jnp.float32)
        return 0

    lax.fori_loop(0, 128, fill, 0)

    for t in range(RPT // 128):
        pltpu.sync_copy(zb, acc.at[pl.ds(s * RPT + t * 128, 128), :])
    plsc.subcore_barrier()

    ebase = wid * EPW
    sems = (sem0, sem1)
    srcv = (srcv0, srcv1)
    dstv = (dstv0, dstv1)
    rows = (rows0, rows1)

    def start(j, b):
        base = pl.multiple_of(ebase + j * C, C)
        pltpu.sync_copy(src_hbm.at[pl.ds(base, C)], srcv[b])
        pltpu.sync_copy(dst_hbm.at[pl.ds(base, C)], dstv[b])
        pltpu.async_copy(tab_hbm.at[srcv[b]], rows[b], sems[b])

    def finish(j, b):
        pltpu.make_async_copy(
            tab_hbm.at[srcv[b]], rows[b], sems[b]
        ).wait()
        pltpu.sync_copy(rows[b], acc.at[dstv[b]], add=True)

    start(0, 0)

    def body(jj, _):
        j0 = 2 * jj
        start(j0 + 1, 1)
        finish(j0, 0)
        start(j0 + 2, 0)
        finish(j0 + 1, 1)
        return 0

    lax.fori_loop(0, (CH - 3) // 2, body, 0)
    j0 = CH - 3
    start(j0 + 1, 1)
    finish(j0, 0)
    start(j0 + 2, 0)
    finish(j0 + 1, 1)
    finish(j0 + 2, 0)
    plsc.subcore_barrier()
    for t in range(RPT // 128):
        pltpu.sync_copy(
            acc.at[pl.ds(s * RPT + t * 128, 128), :],
            out_hbm.at[c, pl.ds(s * RPT + t * 128, 128), :],
        )


BR = 1024


def _scale_body(degp_ref, x_ref, dinv_ref, h0_ref):
    deg = degp_ref[0, :, 0:1] + degp_ref[1, :, 0:1]
    dinv = lax.rsqrt(jnp.maximum(deg, 1.0))
    dinv_ref[...] = dinv
    h0_ref[...] = x_ref[...] * dinv


def _combine_body(q_ref, dinv_ref, tx1_ref, h1_ref):
    dinv = dinv_ref[...]
    tx1 = -dinv * (q_ref[0] + q_ref[1])
    tx1_ref[...] = tx1
    h1_ref[...] = dinv * tx1


def _final_body(x_ref, tx1_ref, r_ref, dinv_ref, fc_ref, w_ref, b_ref, out_ref):
    tx2 = -2.0 * dinv_ref[...] * (r_ref[0] + r_ref[1]) - x_ref[...]
    acc = jnp.dot(fc_ref[:, 0:1] * x_ref[...], w_ref[0],
                  preferred_element_type=jnp.float32)
    acc = acc + jnp.dot(fc_ref[:, 1:2] * tx1_ref[...], w_ref[1],
                        preferred_element_type=jnp.float32)
    acc = acc + jnp.dot(fc_ref[:, 2:3] * tx2, w_ref[2],
                        preferred_element_type=jnp.float32)
    out_ref[...] = acc + b_ref[...]


def _row_spec(width):
    return pl.BlockSpec((BR, width), lambda i: (i, 0))


def _part_spec(width):
    return pl.BlockSpec((NC, BR, width), lambda i: (0, i, 0))


_scale_call = pl.pallas_call(
    _scale_body,
    grid=(NP // BR,),
    in_specs=[_part_spec(DEGW), _row_spec(D)],
    out_specs=[_row_spec(1), _row_spec(D)],
    out_shape=[
        jax.ShapeDtypeStruct((NP, 1), jnp.float32),
        jax.ShapeDtypeStruct((NP, D), jnp.float32),
    ],
)

_combine_call = pl.pallas_call(
    _combine_body,
    grid=(NP // BR,),
    in_specs=[_part_spec(D), _row_spec(1)],
    out_specs=[_row_spec(D), _row_spec(D)],
    out_shape=[
        jax.ShapeDtypeStruct((NP, D), jnp.float32),
        jax.ShapeDtypeStruct((NP, D), jnp.float32),
    ],
)

_final_call = pl.pallas_call(
    _final_body,
    grid=(NP // BR,),
    in_specs=[
        _row_spec(D),
        _row_spec(D),
        _part_spec(D),
        _row_spec(1),
        _row_spec(K),
        pl.BlockSpec((K, D, D), lambda i: (0, 0, 0)),
        pl.BlockSpec((1, D), lambda i: (0, 0)),
    ],
    out_specs=_row_spec(D),
    out_shape=jax.ShapeDtypeStruct((NP, D), jnp.float32),
)


def kernel(x, edge_index, filter_coeff, W, b):
    src = edge_index[0].astype(jnp.int32)
    dst = edge_index[1].astype(jnp.int32)
    x_p = jnp.pad(x, ((0, NP - N), (0, 0)))
    fc_p = jnp.pad(filter_coeff.T, ((0, NP - N), (0, 0)))

    degp = _deg_kernel(dst)
    dinv, h0 = _scale_call(degp, x_p)
    q = _spmv_kernel(src, dst, h0)
    tx1, h1 = _combine_call(q, dinv)
    r = _spmv_kernel(src, dst, h1)
    out_p = _final_call(x_p, tx1, r, dinv, fc_p, W, b[None, :])
    return out_p[:N]


# speedup vs baseline: 24.0930x; 1.5114x over previous
import functools

import jax
import jax.numpy as jnp
from jax import lax
from jax.experimental import pallas as pl
from jax.experimental.pallas import tpu as pltpu
from jax.experimental.pallas import tpu_sc as plsc

N = 10000
D = 128
K = 3
E = 320000

NC = 2
NS = 16
NW = NC * NS
NP = 10240
RPT = NP // NW
EPW = E // NW
C = 80
CH = EPW // C
DEGW = 16

_mesh = plsc.VectorSubcoreMesh(
    core_axis_name="c", subcore_axis_name="s", num_cores=NC, num_subcores=NS
)


@functools.partial(
    pl.kernel,
    out_type=jax.ShapeDtypeStruct((NC, NP, DEGW), jnp.float32),
    mesh=_mesh,
    scratch_types=[
        pltpu.VMEM((EPW,), jnp.int32),
        pltpu.VMEM((C,), jnp.int32),
        pltpu.VMEM((C,), jnp.int32),
        pltpu.VMEM((C, DEGW), jnp.float32),
        pltpu.VMEM((C, DEGW), jnp.float32),
        pltpu.VMEM_SHARED((NP, DEGW), jnp.float32),
        pltpu.SemaphoreType.DMA,
        pltpu.SemaphoreType.DMA,
        pltpu.SemaphoreType.DMA,
    ],
)
def _deg_kernel(dst_hbm, out_hbm, dloc, dstc0, dstc1, onesv, zb, acc, semi,
                sem0, sem1):
    c = lax.axis_index("c")
    s = lax.axis_index("s")
    wid = c * NS + s

    idx_cp = pltpu.async_copy(dst_hbm.at[pl.ds(wid * EPW, EPW)], dloc, semi)

    def fill(i, _):
        zb[i, pl.ds(0, 16)] = jnp.zeros((16,), jnp.float32)
        onesv[i, pl.ds(0, 16)] = jnp.ones((16,), jnp.float32)
        return 0

    lax.fori_loop(0, C, fill, 0)

    for t in range(RPT // C):
        pltpu.sync_copy(zb, acc.at[pl.ds(s * RPT + t * C, C), :])
    idx_cp.wait()
    plsc.subcore_barrier()

    sems = (sem0, sem1)
    dstc = (dstc0, dstc1)

    def start(j, b):
        base = pl.multiple_of(j * C, 16)
        for k in range(C // 16):
            dstc[b][pl.ds(k * 16, 16)] = dloc[pl.ds(base + k * 16, 16)]
        pltpu.async_copy(onesv, acc.at[dstc[b]], sems[b], add=True)

    def drain(b):
        pltpu.make_async_copy(onesv, acc.at[dstc[b]], sems[b]).wait()

    start(0, 0)
    start(1, 1)

    def body(jj, _):
        j0 = 2 * jj
        drain(0)
        start(j0 + 2, 0)
        drain(1)
        start(j0 + 3, 1)
        return 0

    lax.fori_loop(0, (CH - 3) // 2, body, 0)
    drain(0)
    start(CH - 1, 0)
    drain(1)
    drain(0)

    plsc.subcore_barrier()
    pltpu.sync_copy(
        acc.at[pl.ds(s * RPT, RPT), :], out_hbm.at[c, pl.ds(s * RPT, RPT), :]
    )


@functools.partial(
    pl.kernel,
    out_type=jax.ShapeDtypeStruct((NC, NP, D), jnp.float32),
    mesh=_mesh,
    scratch_types=[
        pltpu.VMEM((EPW,), jnp.int32),
        pltpu.VMEM((EPW,), jnp.int32),
        pltpu.VMEM((C,), jnp.int32),
        pltpu.VMEM((C,), jnp.int32),
        pltpu.VMEM((C, D), jnp.float32),
        pltpu.VMEM((C, D), jnp.float32),
        pltpu.VMEM_SHARED((NP, D), jnp.float32),
        pltpu.SemaphoreType.DMA,
        pltpu.SemaphoreType.DMA,
        pltpu.SemaphoreType.DMA,
    ],
)
def _spmv_kernel(src_hbm, dst_hbm, tab_hbm, out_hbm, sloc, dloc, dstc0, dstc1,
                 rows0, rows1, acc, semi, sem0, sem1):
    c = lax.axis_index("c")
    s = lax.axis_index("s")
    wid = c * NS + s

    cp_s = pltpu.async_copy(src_hbm.at[pl.ds(wid * EPW, EPW)], sloc, semi)
    cp_d = pltpu.async_copy(dst_hbm.at[pl.ds(wid * EPW, EPW)], dloc, semi)

    def zfill(i, _):
        for k in range(D // 16):
            rows0[i, pl.ds(k * 16, 16)] = jnp.zeros((16,), jnp.float32)
        return 0

    lax.fori_loop(0, C, zfill, 0)

    for t in range(RPT // C):
        pltpu.sync_copy(rows0, acc.at[pl.ds(s * RPT + t * C, C), :])
    cp_s.wait()
    cp_d.wait()
    plsc.subcore_barrier()

    sems = (sem0, sem1)
    rows = (rows0, rows1)
    dstc = (dstc0, dstc1)

    def start(j, b):
        base = pl.multiple_of(j * C, 16)
        pltpu.async_copy(tab_hbm.at[sloc.at[pl.ds(base, C)]], rows[b], sems[b])
        for k in range(C // 16):
            dstc[b][pl.ds(k * 16, 16)] = dloc[pl.ds(base + k * 16, 16)]

    def finish(j, b):
        pltpu.make_async_copy(
            tab_hbm.at[sloc.at[pl.ds(0, C)]], rows[b], sems[b]
        ).wait()
        pltpu.sync_copy(rows[b], acc.at[dstc[b]], add=True)

    start(0, 0)

    def body(jj, _):
        j0 = 2 * jj
        start(j0 + 1, 1)
        finish(j0, 0)
        start(j0 + 2, 0)
        finish(j0 + 1, 1)
        return 0

    lax.fori_loop(0, (CH - 3) // 2, body, 0)
    j0 = CH - 3
    start(j0 + 1, 1)
    finish(j0, 0)
    start(j0 + 2, 0)
    finish(j0 + 1, 1)
    finish(j0 + 2, 0)
    plsc.subcore_barrier()
    for t in range(RPT // 128):
        pltpu.sync_copy(
            acc.at[pl.ds(s * RPT + t * 128, 128), :],
            out_hbm.at[c, pl.ds(s * RPT + t * 128, 128), :],
        )


BR = 1024


def _scale_body(degp_ref, x_ref, dinv_ref, h0_ref):
    deg = degp_ref[0, :, 0:1] + degp_ref[1, :, 0:1]
    dinv = lax.rsqrt(jnp.maximum(deg, 1.0))
    dinv_ref[...] = dinv
    h0_ref[...] = x_ref[...] * dinv


def _combine_body(q_ref, dinv_ref, tx1_ref, h1_ref):
    dinv = dinv_ref[...]
    tx1 = -dinv * (q_ref[0] + q_ref[1])
    tx1_ref[...] = tx1
    h1_ref[...] = dinv * tx1


def _final_body(x_ref, tx1_ref, r_ref, dinv_ref, fc_ref, w_ref, b_ref, out_ref):
    tx2 = -2.0 * dinv_ref[...] * (r_ref[0] + r_ref[1]) - x_ref[...]
    acc = jnp.dot(fc_ref[:, 0:1] * x_ref[...], w_ref[0],
                  preferred_element_type=jnp.float32)
    acc = acc + jnp.dot(fc_ref[:, 1:2] * tx1_ref[...], w_ref[1],
                        preferred_element_type=jnp.float32)
    acc = acc + jnp.dot(fc_ref[:, 2:3] * tx2, w_ref[2],
                        preferred_element_type=jnp.float32)
    out_ref[...] = acc + b_ref[...]


def _row_spec(width):
    return pl.BlockSpec((BR, width), lambda i: (i, 0))


def _part_spec(width):
    return pl.BlockSpec((NC, BR, width), lambda i: (0, i, 0))


_scale_call = pl.pallas_call(
    _scale_body,
    grid=(NP // BR,),
    in_specs=[_part_spec(DEGW), _row_spec(D)],
    out_specs=[_row_spec(1), _row_spec(D)],
    out_shape=[
        jax.ShapeDtypeStruct((NP, 1), jnp.float32),
        jax.ShapeDtypeStruct((NP, D), jnp.float32),
    ],
)

_combine_call = pl.pallas_call(
    _combine_body,
    grid=(NP // BR,),
    in_specs=[_part_spec(D), _row_spec(1)],
    out_specs=[_row_spec(D), _row_spec(D)],
    out_shape=[
        jax.ShapeDtypeStruct((NP, D), jnp.float32),
        jax.ShapeDtypeStruct((NP, D), jnp.float32),
    ],
)

_final_call = pl.pallas_call(
    _final_body,
    grid=(NP // BR,),
    in_specs=[
        _row_spec(D),
        _row_spec(D),
        _part_spec(D),
        _row_spec(1),
        _row_spec(K),
        pl.BlockSpec((K, D, D), lambda i: (0, 0, 0)),
        pl.BlockSpec((1, D), lambda i: (0, 0)),
    ],
    out_specs=_row_spec(D),
    out_shape=jax.ShapeDtypeStruct((NP, D), jnp.float32),
)


def kernel(x, edge_index, filter_coeff, W, b):
    src = edge_index[0].astype(jnp.int32)
    dst = edge_index[1].astype(jnp.int32)
    x_p = jnp.pad(x, ((0, NP - N), (0, 0)))
    fc_p = jnp.pad(filter_coeff.T, ((0, NP - N), (0, 0)))

    degp = _deg_kernel(dst)
    dinv, h0 = _scale_call(degp, x_p)
    q = _spmv_kernel(src, dst, h0)
    tx1, h1 = _combine_call(q, dinv)
    r = _spmv_kernel(src, dst, h1)
    out_p = _final_call(x_p, tx1, r, dinv, fc_p, W, b[None, :])
    return out_p[:N]
